# baseline (speedup 1.0000x reference)
import jax
import jax.numpy as jnp
from jax.experimental import pallas as pl
from jax.experimental.pallas import tpu as pltpu

_ALPHA = 1.0
_GAMMA = 1e-06

_BT = 8192
_D = 256
_K = 512
_QT = 128


def _place(val, lane):
    r = jax.lax.broadcasted_iota(jnp.int32, (8, 128), 0)
    l = jax.lax.broadcasted_iota(jnp.int32, (8, 128), 1)
    return jnp.where((r == 0) & (l == lane), val, 0.0)


def _fused_step(p_ref, x_ref, q_ref, m_ref, w_ref, dw_ref,
                out_ref, rg_acc, g_acc):
    i = pl.program_id(0)
    nsteps = pl.num_programs(0)

    @pl.when(i == 0)
    def _init():
        rg_acc[...] = jnp.zeros_like(rg_acc)
        g_acc[...] = jnp.zeros_like(g_acc)
        out_ref[...] = jnp.zeros_like(out_ref)

    p = p_ref[...]
    x = x_ref[...]
    q = q_ref[...]
    m = m_ref[...]
    w = w_ref[...]
    dw = dw_ref[...]

    hi = jax.lax.Precision.HIGHEST
    y = jax.lax.dot_general(p, w, (((1,), (1,)), ((), ())),
                            precision=hi, preferred_element_type=jnp.float32)
    e = y - x
    rec = jnp.sum(e * e)
    a = jnp.tanh(y)
    dsum = jnp.sum(a * dw)
    rg_acc[...] += jax.lax.dot_general(e, p, (((0,), (0,)), ((), ())),
                                       precision=hi,
                                       preferred_element_type=jnp.float32)
    g_acc[...] += jax.lax.dot_general(1.0 - a * a, p, (((0,), (0,)), ((), ())),
                                      precision=hi,
                                      preferred_element_type=jnp.float32)

    qb = q.astype(jnp.bfloat16)
    mb = m.astype(jnp.bfloat16)
    qm = jax.lax.dot_general(q, m, (((1,), (0,)), ((), ())),
                             precision=hi, preferred_element_type=jnp.float32)
    msq = jnp.sum(m * m, axis=0, keepdims=True)
    hsq = jnp.sum(q * q)

    _CH = 128
    _NC = _K // _CH
    kio = jax.lax.broadcasted_iota(jnp.int32, (_QT, _CH), 1)
    mv_l, ix_l, vv_l = [], [], []
    for c in range(_NC):
        mbc = mb[:, c * _CH:(c + 1) * _CH]
        acc = jnp.zeros((_QT, _CH), jnp.float32)
        for dd in range(0, _D, 4):
            terms = [jnp.abs(qb[:, dd + j:dd + j + 1] - mbc[dd + j:dd + j + 1, :])
                     for j in range(4)]
            while len(terms) > 1:
                terms = [terms[t] + terms[t + 1]
                         for t in range(0, len(terms), 2)]
            acc = acc + terms[0].astype(jnp.float32)
        mv = jnp.min(acc, axis=1, keepdims=True)
        ix = jnp.min(jnp.where(acc == mv, kio, _K), axis=1, keepdims=True)
        qmc = qm[:, c * _CH:(c + 1) * _CH]
        msqc = msq[:, c * _CH:(c + 1) * _CH]
        vv = jnp.sum(jnp.where(kio == ix, msqc - 2.0 * qmc, 0.0),
                     axis=1, keepdims=True)
        mv_l.append(mv)
        ix_l.append(ix + c * _CH)
        vv_l.append(vv)

    mv_all = jnp.concatenate(mv_l, axis=1)
    ix_all = jnp.concatenate(ix_l, axis=1)
    vv_all = jnp.concatenate(vv_l, axis=1)
    minv = jnp.min(mv_all, axis=1, keepdims=True)
    idx = jnp.min(jnp.where(mv_all == minv, ix_all, _K), axis=1, keepdims=True)
    val = jnp.sum(jnp.where(ix_all == idx, vv_all, 0.0), axis=1)
    msum = hsq + jnp.sum(val)

    out_ref[...] += _place(rec, 0) + _place(dsum, 1) + _place(msum, 2)

    @pl.when(i == nsteps - 1)
    def _fin():
        rg = rg_acc[...]
        g = g_acc[...] * jnp.reshape(dw_ref[...], (_D, 1))
        out_ref[...] += _place(jnp.sum(rg * rg), 3) + _place(jnp.sum(g * g), 4)


def kernel(pre_x, X, H, M, W, disc_w):
    B, T, dx = pre_x.shape
    p = pre_x.reshape(_BT, _D)
    x = X.reshape(_BT, _D)
    q = jnp.transpose(H, (0, 2, 1)).reshape(_BT, _D)
    dw = disc_w.reshape(1, _D)

    nsteps = _BT // _QT
    out = pl.pallas_call(
        _fused_step,
        grid=(nsteps,),
        in_specs=[
            pl.BlockSpec((_QT, _D), lambda i: (i, 0)),
            pl.BlockSpec((_QT, _D), lambda i: (i, 0)),
            pl.BlockSpec((_QT, _D), lambda i: (i, 0)),
            pl.BlockSpec((_D, _K), lambda i: (0, 0)),
            pl.BlockSpec((_D, _D), lambda i: (0, 0)),
            pl.BlockSpec((1, _D), lambda i: (0, 0)),
        ],
        out_specs=pl.BlockSpec((8, 128), lambda i: (0, 0)),
        out_shape=jax.ShapeDtypeStruct((8, 128), jnp.float32),
        scratch_shapes=[
            pltpu.VMEM((_D, _D), jnp.float32),
            pltpu.VMEM((_D, _D), jnp.float32),
        ],
        compiler_params=pltpu.CompilerParams(
            dimension_semantics=("arbitrary",),
            vmem_limit_bytes=100 * 1024 * 1024,
        ),
    )(p, x, q, M, W, dw)

    n_rec = float(_BT * _D)
    loss_rec = out[0, 0] / n_rec
    loss_d = -out[0, 1] / float(_BT)
    loss_m = 2.0 * out[0, 2] / n_rec
    rg_norm = jnp.sqrt(out[0, 3]) * (2.0 / n_rec)
    dg_norm = jnp.sqrt(out[0, 4]) / float(_BT)
    lmbda = rg_norm / (dg_norm + _GAMMA)
    return loss_rec + _ALPHA * loss_m + lmbda * loss_d


# speedup vs baseline: 1.2805x; 1.2805x over previous
import jax
import jax.numpy as jnp
from jax.experimental import pallas as pl
from jax.experimental.pallas import tpu as pltpu

_ALPHA = 1.0
_GAMMA = 1e-06

_BT = 8192
_D = 256
_K = 512
_QT = 256


def _place(val, lane):
    r = jax.lax.broadcasted_iota(jnp.int32, (8, 128), 0)
    l = jax.lax.broadcasted_iota(jnp.int32, (8, 128), 1)
    return jnp.where((r == 0) & (l == lane), val, 0.0)


def _fused_step(p_ref, x_ref, q_ref, m_ref, w_ref, dw_ref,
                out_ref, rg_acc, g_acc):
    i = pl.program_id(0)
    nsteps = pl.num_programs(0)

    @pl.when(i == 0)
    def _init():
        rg_acc[...] = jnp.zeros_like(rg_acc)
        g_acc[...] = jnp.zeros_like(g_acc)
        out_ref[...] = jnp.zeros_like(out_ref)

    p = p_ref[...]
    x = x_ref[...]
    q = q_ref[...]
    m = m_ref[...]
    w = w_ref[...]
    dw = dw_ref[...]

    hi = jax.lax.Precision.HIGHEST
    y = jax.lax.dot_general(p, w, (((1,), (1,)), ((), ())),
                            precision=hi, preferred_element_type=jnp.float32)
    e = y - x
    rec = jnp.sum(e * e)
    a = jnp.tanh(y)
    dsum = jnp.sum(a * dw)
    rg_acc[...] += jax.lax.dot_general(e, p, (((0,), (0,)), ((), ())),
                                       precision=hi,
                                       preferred_element_type=jnp.float32)
    g_acc[...] += jax.lax.dot_general(1.0 - a * a, p, (((0,), (0,)), ((), ())),
                                      precision=hi,
                                      preferred_element_type=jnp.float32)

    qb = q.astype(jnp.bfloat16)
    mb = m.astype(jnp.bfloat16)
    qm = jax.lax.dot_general(q, m, (((1,), (0,)), ((), ())),
                             precision=hi, preferred_element_type=jnp.float32)
    msq = jnp.sum(m * m, axis=0, keepdims=True)
    hsq = jnp.sum(q * q)

    mcs = jnp.sum(m, axis=0, keepdims=True)
    _CH = 128
    _NC = _K // _CH
    kio = jax.lax.broadcasted_iota(jnp.int32, (_QT, _CH), 1)
    mv_l, ix_l, vv_l = [], [], []
    for c in range(_NC):
        mbc = mb[:, c * _CH:(c + 1) * _CH]
        acc = jnp.zeros((_QT, _CH), jnp.float32)
        for dd in range(0, _D, 8):
            terms = [jnp.minimum(qb[:, dd + j:dd + j + 1],
                                 mbc[dd + j:dd + j + 1, :])
                     for j in range(8)]
            while len(terms) > 1:
                terms = [terms[t] + terms[t + 1]
                         for t in range(0, len(terms), 2)]
            acc = acc + terms[0].astype(jnp.float32)
        sc = mcs[:, c * _CH:(c + 1) * _CH] - 2.0 * acc
        mv = jnp.min(sc, axis=1, keepdims=True)
        ix = jnp.min(jnp.where(sc == mv, kio, _K), axis=1, keepdims=True)
        qmc = qm[:, c * _CH:(c + 1) * _CH]
        msqc = msq[:, c * _CH:(c + 1) * _CH]
        vv = jnp.sum(jnp.where(kio == ix, msqc - 2.0 * qmc, 0.0),
                     axis=1, keepdims=True)
        mv_l.append(mv)
        ix_l.append(ix + c * _CH)
        vv_l.append(vv)

    mv_all = jnp.concatenate(mv_l, axis=1)
    ix_all = jnp.concatenate(ix_l, axis=1)
    vv_all = jnp.concatenate(vv_l, axis=1)
    minv = jnp.min(mv_all, axis=1, keepdims=True)
    idx = jnp.min(jnp.where(mv_all == minv, ix_all, _K), axis=1, keepdims=True)
    val = jnp.sum(jnp.where(ix_all == idx, vv_all, 0.0), axis=1)
    msum = hsq + jnp.sum(val)

    out_ref[...] += _place(rec, 0) + _place(dsum, 1) + _place(msum, 2)

    @pl.when(i == nsteps - 1)
    def _fin():
        rg = rg_acc[...]
        g = g_acc[...] * jnp.reshape(dw_ref[...], (_D, 1))
        out_ref[...] += _place(jnp.sum(rg * rg), 3) + _place(jnp.sum(g * g), 4)


def kernel(pre_x, X, H, M, W, disc_w):
    B, T, dx = pre_x.shape
    p = pre_x.reshape(_BT, _D)
    x = X.reshape(_BT, _D)
    q = jnp.transpose(H, (0, 2, 1)).reshape(_BT, _D)
    dw = disc_w.reshape(1, _D)

    nsteps = _BT // _QT
    out = pl.pallas_call(
        _fused_step,
        grid=(nsteps,),
        in_specs=[
            pl.BlockSpec((_QT, _D), lambda i: (i, 0)),
            pl.BlockSpec((_QT, _D), lambda i: (i, 0)),
            pl.BlockSpec((_QT, _D), lambda i: (i, 0)),
            pl.BlockSpec((_D, _K), lambda i: (0, 0)),
            pl.BlockSpec((_D, _D), lambda i: (0, 0)),
            pl.BlockSpec((1, _D), lambda i: (0, 0)),
        ],
        out_specs=pl.BlockSpec((8, 128), lambda i: (0, 0)),
        out_shape=jax.ShapeDtypeStruct((8, 128), jnp.float32),
        scratch_shapes=[
            pltpu.VMEM((_D, _D), jnp.float32),
            pltpu.VMEM((_D, _D), jnp.float32),
        ],
        compiler_params=pltpu.CompilerParams(
            dimension_semantics=("arbitrary",),
            vmem_limit_bytes=100 * 1024 * 1024,
        ),
    )(p, x, q, M, W, dw)

    n_rec = float(_BT * _D)
    loss_rec = out[0, 0] / n_rec
    loss_d = -out[0, 1] / float(_BT)
    loss_m = 2.0 * out[0, 2] / n_rec
    rg_norm = jnp.sqrt(out[0, 3]) * (2.0 / n_rec)
    dg_norm = jnp.sqrt(out[0, 4]) / float(_BT)
    lmbda = rg_norm / (dg_norm + _GAMMA)
    return loss_rec + _ALPHA * loss_m + lmbda * loss_d


# speedup vs baseline: 1.3337x; 1.0416x over previous
import jax
import jax.numpy as jnp
from jax.experimental import pallas as pl
from jax.experimental.pallas import tpu as pltpu

_ALPHA = 1.0
_GAMMA = 1e-06

_BT = 8192
_D = 256
_K = 512
_QT = 256


def _place(val, lane):
    r = jax.lax.broadcasted_iota(jnp.int32, (8, 128), 0)
    l = jax.lax.broadcasted_iota(jnp.int32, (8, 128), 1)
    return jnp.where((r == 0) & (l == lane), val, 0.0)


def _fused_step(p_ref, x_ref, q_ref, m_ref, w_ref, dw_ref,
                out_ref, rg_acc, g_acc, mb_ref, mstat_ref):
    i = pl.program_id(0)
    nsteps = pl.num_programs(0)

    @pl.when(i == 0)
    def _init():
        rg_acc[...] = jnp.zeros_like(rg_acc)
        g_acc[...] = jnp.zeros_like(g_acc)
        out_ref[...] = jnp.zeros_like(out_ref)
        m0 = m_ref[...]
        mb_ref[...] = m0.astype(jnp.bfloat16)
        mstat_ref[0:1, :] = jnp.sum(m0 * m0, axis=0, keepdims=True)
        mstat_ref[1:2, :] = jnp.sum(m0, axis=0, keepdims=True)

    p = p_ref[...]
    x = x_ref[...]
    q = q_ref[...]
    m = m_ref[...]
    w = w_ref[...]
    dw = dw_ref[...]

    hi = jax.lax.Precision.HIGHEST
    y = jax.lax.dot_general(p, w, (((1,), (1,)), ((), ())),
                            precision=hi, preferred_element_type=jnp.float32)
    e = y - x
    rec = jnp.sum(e * e)
    a = jnp.tanh(y)
    dsum = jnp.sum(a * dw)
    rg_acc[...] += jax.lax.dot_general(e, p, (((0,), (0,)), ((), ())),
                                       precision=hi,
                                       preferred_element_type=jnp.float32)
    g_acc[...] += jax.lax.dot_general(1.0 - a * a, p, (((0,), (0,)), ((), ())),
                                      precision=hi,
                                      preferred_element_type=jnp.float32)

    qb = q.astype(jnp.bfloat16)
    mb = mb_ref[...]
    qm = jax.lax.dot_general(q, m, (((1,), (0,)), ((), ())),
                             precision=hi, preferred_element_type=jnp.float32)
    msq = mstat_ref[0:1, :]
    hsq = jnp.sum(q * q)

    mcs = mstat_ref[1:2, :]
    _CH = 128
    _NC = _K // _CH
    kio = jax.lax.broadcasted_iota(jnp.int32, (_QT, _CH), 1)
    mv_l, ix_l, vv_l = [], [], []
    for c in range(_NC):
        mbc = mb[:, c * _CH:(c + 1) * _CH]
        acc = jnp.zeros((_QT, _CH), jnp.float32)
        for dd in range(0, _D, 16):
            terms = [jnp.minimum(qb[:, dd + j:dd + j + 1],
                                 mbc[dd + j:dd + j + 1, :])
                     for j in range(16)]
            while len(terms) > 1:
                terms = [terms[t] + terms[t + 1]
                         for t in range(0, len(terms), 2)]
            acc = acc + terms[0].astype(jnp.float32)
        sc = mcs[:, c * _CH:(c + 1) * _CH] - 2.0 * acc
        mv = jnp.min(sc, axis=1, keepdims=True)
        ix = jnp.min(jnp.where(sc == mv, kio, _K), axis=1, keepdims=True)
        qmc = qm[:, c * _CH:(c + 1) * _CH]
        msqc = msq[:, c * _CH:(c + 1) * _CH]
        vv = jnp.sum(jnp.where(kio == ix, msqc - 2.0 * qmc, 0.0),
                     axis=1, keepdims=True)
        mv_l.append(mv)
        ix_l.append(ix + c * _CH)
        vv_l.append(vv)

    mv_all = jnp.concatenate(mv_l, axis=1)
    ix_all = jnp.concatenate(ix_l, axis=1)
    vv_all = jnp.concatenate(vv_l, axis=1)
    minv = jnp.min(mv_all, axis=1, keepdims=True)
    idx = jnp.min(jnp.where(mv_all == minv, ix_all, _K), axis=1, keepdims=True)
    val = jnp.sum(jnp.where(ix_all == idx, vv_all, 0.0), axis=1)
    msum = hsq + jnp.sum(val)

    out_ref[...] += _place(rec, 0) + _place(dsum, 1) + _place(msum, 2)

    @pl.when(i == nsteps - 1)
    def _fin():
        rg = rg_acc[...]
        g = g_acc[...] * jnp.reshape(dw_ref[...], (_D, 1))
        out_ref[...] += _place(jnp.sum(rg * rg), 3) + _place(jnp.sum(g * g), 4)


def kernel(pre_x, X, H, M, W, disc_w):
    B, T, dx = pre_x.shape
    p = pre_x.reshape(_BT, _D)
    x = X.reshape(_BT, _D)
    q = jnp.transpose(H, (0, 2, 1)).reshape(_BT, _D)
    dw = disc_w.reshape(1, _D)

    nsteps = _BT // _QT
    out = pl.pallas_call(
        _fused_step,
        grid=(nsteps,),
        in_specs=[
            pl.BlockSpec((_QT, _D), lambda i: (i, 0)),
            pl.BlockSpec((_QT, _D), lambda i: (i, 0)),
            pl.BlockSpec((_QT, _D), lambda i: (i, 0)),
            pl.BlockSpec((_D, _K), lambda i: (0, 0)),
            pl.BlockSpec((_D, _D), lambda i: (0, 0)),
            pl.BlockSpec((1, _D), lambda i: (0, 0)),
        ],
        out_specs=pl.BlockSpec((8, 128), lambda i: (0, 0)),
        out_shape=jax.ShapeDtypeStruct((8, 128), jnp.float32),
        scratch_shapes=[
            pltpu.VMEM((_D, _D), jnp.float32),
            pltpu.VMEM((_D, _D), jnp.float32),
            pltpu.VMEM((_D, _K), jnp.bfloat16),
            pltpu.VMEM((8, _K), jnp.float32),
        ],
        compiler_params=pltpu.CompilerParams(
            dimension_semantics=("arbitrary",),
            vmem_limit_bytes=100 * 1024 * 1024,
        ),
    )(p, x, q, M, W, dw)

    n_rec = float(_BT * _D)
    loss_rec = out[0, 0] / n_rec
    loss_d = -out[0, 1] / float(_BT)
    loss_m = 2.0 * out[0, 2] / n_rec
    rg_norm = jnp.sqrt(out[0, 3]) * (2.0 / n_rec)
    dg_norm = jnp.sqrt(out[0, 4]) / float(_BT)
    lmbda = rg_norm / (dg_norm + _GAMMA)
    return loss_rec + _ALPHA * loss_m + lmbda * loss_d


# speedup vs baseline: 1.3403x; 1.0050x over previous
import jax
import jax.numpy as jnp
from jax.experimental import pallas as pl
from jax.experimental.pallas import tpu as pltpu

_ALPHA = 1.0
_GAMMA = 1e-06

_BT = 8192
_D = 256
_K = 512
_QT = 256


def _place(val, lane):
    r = jax.lax.broadcasted_iota(jnp.int32, (8, 128), 0)
    l = jax.lax.broadcasted_iota(jnp.int32, (8, 128), 1)
    return jnp.where((r == 0) & (l == lane), val, 0.0)


def _fused_step(p_ref, x_ref, q_ref, m_ref, w_ref, dw_ref,
                out_ref, rg_acc, g_acc, mb_ref, mstat_ref):
    i = pl.program_id(0)
    nsteps = pl.num_programs(0)

    @pl.when(i == 0)
    def _init():
        rg_acc[...] = jnp.zeros_like(rg_acc)
        g_acc[...] = jnp.zeros_like(g_acc)
        out_ref[...] = jnp.zeros_like(out_ref)
        m0 = m_ref[...]
        mb_ref[...] = m0.astype(jnp.bfloat16)
        mstat_ref[0:1, :] = jnp.sum(m0 * m0, axis=0, keepdims=True)
        mstat_ref[1:2, :] = jnp.sum(m0, axis=0, keepdims=True)

    p = p_ref[...]
    x = x_ref[...]
    q = q_ref[...]
    m = m_ref[...]
    w = w_ref[...]
    dw = dw_ref[...]

    hi = jax.lax.Precision.HIGHEST
    y = jax.lax.dot_general(p, w, (((1,), (1,)), ((), ())),
                            precision=hi, preferred_element_type=jnp.float32)
    e = y - x
    rec = jnp.sum(e * e)
    a = jnp.tanh(y)
    dsum = jnp.sum(a * dw)
    rg_acc[...] += jax.lax.dot_general(e, p, (((0,), (0,)), ((), ())),
                                       precision=hi,
                                       preferred_element_type=jnp.float32)
    g_acc[...] += jax.lax.dot_general(1.0 - a * a, p, (((0,), (0,)), ((), ())),
                                      precision=hi,
                                      preferred_element_type=jnp.float32)

    qb = q.astype(jnp.bfloat16)
    mb = mb_ref[...]
    qm = jax.lax.dot_general(q, m, (((1,), (0,)), ((), ())),
                             precision=hi, preferred_element_type=jnp.float32)
    msq = mstat_ref[0:1, :]
    hsq = jnp.sum(q * q)

    mcs = mstat_ref[1:2, :]
    _CH = 128
    _NC = _K // _CH
    kio = jax.lax.broadcasted_iota(jnp.int32, (_QT, _CH), 1)
    mv_l, ix_l, vv_l = [], [], []
    for c in range(_NC):
        mbc = mb[:, c * _CH:(c + 1) * _CH]
        acc = jnp.zeros((_QT, _CH), jnp.float32)
        for dd in range(0, _D, 32):
            terms = [jnp.minimum(qb[:, dd + j:dd + j + 1],
                                 mbc[dd + j:dd + j + 1, :])
                     for j in range(32)]
            while len(terms) > 1:
                terms = [terms[t] + terms[t + 1]
                         for t in range(0, len(terms), 2)]
            acc = acc + terms[0].astype(jnp.float32)
        sc = mcs[:, c * _CH:(c + 1) * _CH] - 2.0 * acc
        mv = jnp.min(sc, axis=1, keepdims=True)
        ix = jnp.min(jnp.where(sc == mv, kio, _K), axis=1, keepdims=True)
        qmc = qm[:, c * _CH:(c + 1) * _CH]
        msqc = msq[:, c * _CH:(c + 1) * _CH]
        vv = jnp.sum(jnp.where(kio == ix, msqc - 2.0 * qmc, 0.0),
                     axis=1, keepdims=True)
        mv_l.append(mv)
        ix_l.append(ix + c * _CH)
        vv_l.append(vv)

    mv_all = jnp.concatenate(mv_l, axis=1)
    ix_all = jnp.concatenate(ix_l, axis=1)
    vv_all = jnp.concatenate(vv_l, axis=1)
    minv = jnp.min(mv_all, axis=1, keepdims=True)
    idx = jnp.min(jnp.where(mv_all == minv, ix_all, _K), axis=1, keepdims=True)
    val = jnp.sum(jnp.where(ix_all == idx, vv_all, 0.0), axis=1)
    msum = hsq + jnp.sum(val)

    out_ref[...] += _place(rec, 0) + _place(dsum, 1) + _place(msum, 2)

    @pl.when(i == nsteps - 1)
    def _fin():
        rg = rg_acc[...]
        g = g_acc[...] * jnp.reshape(dw_ref[...], (_D, 1))
        out_ref[...] += _place(jnp.sum(rg * rg), 3) + _place(jnp.sum(g * g), 4)


def kernel(pre_x, X, H, M, W, disc_w):
    B, T, dx = pre_x.shape
    p = pre_x.reshape(_BT, _D)
    x = X.reshape(_BT, _D)
    q = jnp.transpose(H, (0, 2, 1)).reshape(_BT, _D)
    dw = disc_w.reshape(1, _D)

    nsteps = _BT // _QT
    out = pl.pallas_call(
        _fused_step,
        grid=(nsteps,),
        in_specs=[
            pl.BlockSpec((_QT, _D), lambda i: (i, 0)),
            pl.BlockSpec((_QT, _D), lambda i: (i, 0)),
            pl.BlockSpec((_QT, _D), lambda i: (i, 0)),
            pl.BlockSpec((_D, _K), lambda i: (0, 0)),
            pl.BlockSpec((_D, _D), lambda i: (0, 0)),
            pl.BlockSpec((1, _D), lambda i: (0, 0)),
        ],
        out_specs=pl.BlockSpec((8, 128), lambda i: (0, 0)),
        out_shape=jax.ShapeDtypeStruct((8, 128), jnp.float32),
        scratch_shapes=[
            pltpu.VMEM((_D, _D), jnp.float32),
            pltpu.VMEM((_D, _D), jnp.float32),
            pltpu.VMEM((_D, _K), jnp.bfloat16),
            pltpu.VMEM((8, _K), jnp.float32),
        ],
        compiler_params=pltpu.CompilerParams(
            dimension_semantics=("arbitrary",),
            vmem_limit_bytes=100 * 1024 * 1024,
        ),
    )(p, x, q, M, W, dw)

    n_rec = float(_BT * _D)
    loss_rec = out[0, 0] / n_rec
    loss_d = -out[0, 1] / float(_BT)
    loss_m = 2.0 * out[0, 2] / n_rec
    rg_norm = jnp.sqrt(out[0, 3]) * (2.0 / n_rec)
    dg_norm = jnp.sqrt(out[0, 4]) / float(_BT)
    lmbda = rg_norm / (dg_norm + _GAMMA)
    return loss_rec + _ALPHA * loss_m + lmbda * loss_d


# speedup vs baseline: 1.3410x; 1.0005x over previous
import jax
import jax.numpy as jnp
from jax.experimental import pallas as pl
from jax.experimental.pallas import tpu as pltpu

_ALPHA = 1.0
_GAMMA = 1e-06

_BT = 8192
_D = 256
_K = 512
_QT = 256


def _place(val, lane):
    r = jax.lax.broadcasted_iota(jnp.int32, (8, 128), 0)
    l = jax.lax.broadcasted_iota(jnp.int32, (8, 128), 1)
    return jnp.where((r == 0) & (l == lane), val, 0.0)


def _fused_step(p_ref, x_ref, q_ref, m_ref, w_ref, dw_ref,
                out_ref, rg_acc, g_acc, mb_ref, mstat_ref):
    i = pl.program_id(0)
    nsteps = pl.num_programs(0)

    @pl.when(i == 0)
    def _init():
        rg_acc[...] = jnp.zeros_like(rg_acc)
        g_acc[...] = jnp.zeros_like(g_acc)
        out_ref[...] = jnp.zeros_like(out_ref)
        m0 = m_ref[...]
        mb_ref[...] = m0.astype(jnp.bfloat16)
        mstat_ref[0:1, :] = jnp.sum(m0 * m0, axis=0, keepdims=True)
        mstat_ref[1:2, :] = jnp.sum(m0, axis=0, keepdims=True)

    p = p_ref[...]
    x = x_ref[...]
    q = q_ref[...]
    m = m_ref[...]
    w = w_ref[...]
    dw = dw_ref[...]

    hi = jax.lax.Precision.HIGHEST
    y = jax.lax.dot_general(p, w, (((1,), (1,)), ((), ())),
                            precision=hi, preferred_element_type=jnp.float32)
    e = y - x
    rec = jnp.sum(e * e)
    a = jnp.tanh(y)
    dsum = jnp.sum(a * dw)
    rg_acc[...] += jax.lax.dot_general(e, p, (((0,), (0,)), ((), ())),
                                       precision=hi,
                                       preferred_element_type=jnp.float32)
    g_acc[...] += jax.lax.dot_general(1.0 - a * a, p, (((0,), (0,)), ((), ())),
                                      precision=hi,
                                      preferred_element_type=jnp.float32)

    qb = q.astype(jnp.bfloat16)
    mb = mb_ref[...]
    qm = jax.lax.dot_general(q, m, (((1,), (0,)), ((), ())),
                             precision=hi, preferred_element_type=jnp.float32)
    msq = mstat_ref[0:1, :]
    hsq = jnp.sum(q * q)

    mcs = mstat_ref[1:2, :]
    _CH = 128
    _NC = _K // _CH
    kio = jax.lax.broadcasted_iota(jnp.int32, (_QT, _CH), 1)
    mv_l, ix_l, vv_l = [], [], []
    for c0 in range(0, _NC, 2):
        accs = [jnp.zeros((_QT, _CH), jnp.float32) for _ in range(2)]
        for dd in range(0, _D, 32):
            cols = [qb[:, dd + j:dd + j + 1] for j in range(32)]
            for ci in range(2):
                c = c0 + ci
                mbc = mb[:, c * _CH:(c + 1) * _CH]
                terms = [jnp.minimum(cols[j], mbc[dd + j:dd + j + 1, :])
                         for j in range(32)]
                while len(terms) > 1:
                    terms = [terms[t] + terms[t + 1]
                             for t in range(0, len(terms), 2)]
                accs[ci] = accs[ci] + terms[0].astype(jnp.float32)
        for ci in range(2):
            c = c0 + ci
            sc = mcs[:, c * _CH:(c + 1) * _CH] - 2.0 * accs[ci]
            mv = jnp.min(sc, axis=1, keepdims=True)
            ix = jnp.min(jnp.where(sc == mv, kio, _K), axis=1, keepdims=True)
            qmc = qm[:, c * _CH:(c + 1) * _CH]
            msqc = msq[:, c * _CH:(c + 1) * _CH]
            vv = jnp.sum(jnp.where(kio == ix, msqc - 2.0 * qmc, 0.0),
                         axis=1, keepdims=True)
            mv_l.append(mv)
            ix_l.append(ix + c * _CH)
            vv_l.append(vv)

    mv_all = jnp.concatenate(mv_l, axis=1)
    ix_all = jnp.concatenate(ix_l, axis=1)
    vv_all = jnp.concatenate(vv_l, axis=1)
    minv = jnp.min(mv_all, axis=1, keepdims=True)
    idx = jnp.min(jnp.where(mv_all == minv, ix_all, _K), axis=1, keepdims=True)
    val = jnp.sum(jnp.where(ix_all == idx, vv_all, 0.0), axis=1)
    msum = hsq + jnp.sum(val)

    out_ref[...] += _place(rec, 0) + _place(dsum, 1) + _place(msum, 2)

    @pl.when(i == nsteps - 1)
    def _fin():
        rg = rg_acc[...]
        g = g_acc[...] * jnp.reshape(dw_ref[...], (_D, 1))
        out_ref[...] += _place(jnp.sum(rg * rg), 3) + _place(jnp.sum(g * g), 4)


def kernel(pre_x, X, H, M, W, disc_w):
    B, T, dx = pre_x.shape
    p = pre_x.reshape(_BT, _D)
    x = X.reshape(_BT, _D)
    q = jnp.transpose(H, (0, 2, 1)).reshape(_BT, _D)
    dw = disc_w.reshape(1, _D)

    nsteps = _BT // _QT
    out = pl.pallas_call(
        _fused_step,
        grid=(nsteps,),
        in_specs=[
            pl.BlockSpec((_QT, _D), lambda i: (i, 0)),
            pl.BlockSpec((_QT, _D), lambda i: (i, 0)),
            pl.BlockSpec((_QT, _D), lambda i: (i, 0)),
            pl.BlockSpec((_D, _K), lambda i: (0, 0)),
            pl.BlockSpec((_D, _D), lambda i: (0, 0)),
            pl.BlockSpec((1, _D), lambda i: (0, 0)),
        ],
        out_specs=pl.BlockSpec((8, 128), lambda i: (0, 0)),
        out_shape=jax.ShapeDtypeStruct((8, 128), jnp.float32),
        scratch_shapes=[
            pltpu.VMEM((_D, _D), jnp.float32),
            pltpu.VMEM((_D, _D), jnp.float32),
            pltpu.VMEM((_D, _K), jnp.bfloat16),
            pltpu.VMEM((8, _K), jnp.float32),
        ],
        compiler_params=pltpu.CompilerParams(
            dimension_semantics=("arbitrary",),
            vmem_limit_bytes=100 * 1024 * 1024,
        ),
    )(p, x, q, M, W, dw)

    n_rec = float(_BT * _D)
    loss_rec = out[0, 0] / n_rec
    loss_d = -out[0, 1] / float(_BT)
    loss_m = 2.0 * out[0, 2] / n_rec
    rg_norm = jnp.sqrt(out[0, 3]) * (2.0 / n_rec)
    dg_norm = jnp.sqrt(out[0, 4]) / float(_BT)
    lmbda = rg_norm / (dg_norm + _GAMMA)
    return loss_rec + _ALPHA * loss_m + lmbda * loss_d


# speedup vs baseline: 1.4806x; 1.1041x over previous
import jax
import jax.numpy as jnp
from jax.experimental import pallas as pl
from jax.experimental.pallas import tpu as pltpu

_ALPHA = 1.0
_GAMMA = 1e-06

_BT = 8192
_D = 256
_K = 512
_QT = 256


def _place(val, lane):
    r = jax.lax.broadcasted_iota(jnp.int32, (8, 128), 0)
    l = jax.lax.broadcasted_iota(jnp.int32, (8, 128), 1)
    return jnp.where((r == 0) & (l == lane), val, 0.0)


def _fused_step(p_ref, x_ref, q_ref, m_ref, w_ref, dw_ref,
                out_ref, rg_acc, g_acc, mb_ref, mstat_ref):
    i = pl.program_id(0)
    nsteps = pl.num_programs(0)

    @pl.when(i == 0)
    def _init():
        rg_acc[...] = jnp.zeros_like(rg_acc)
        g_acc[...] = jnp.zeros_like(g_acc)
        out_ref[...] = jnp.zeros_like(out_ref)
        m0 = m_ref[...]
        mb_ref[...] = m0.astype(jnp.bfloat16)
        mstat_ref[0:1, :] = jnp.sum(m0 * m0, axis=0, keepdims=True)
        mstat_ref[1:2, :] = jnp.sum(m0, axis=0, keepdims=True)

    p = p_ref[...]
    x = x_ref[...]
    qt = q_ref[0]
    q = qt.T
    m = m_ref[...]
    w = w_ref[...]
    dw = dw_ref[...]

    hi = jax.lax.Precision.HIGHEST
    y = jax.lax.dot_general(p, w, (((1,), (1,)), ((), ())),
                            precision=hi, preferred_element_type=jnp.float32)
    e = y - x
    rec = jnp.sum(e * e)
    a = jnp.tanh(y)
    dsum = jnp.sum(a * dw)
    rg_acc[...] += jax.lax.dot_general(e, p, (((0,), (0,)), ((), ())),
                                       precision=hi,
                                       preferred_element_type=jnp.float32)
    g_acc[...] += jax.lax.dot_general(1.0 - a * a, p, (((0,), (0,)), ((), ())),
                                      precision=hi,
                                      preferred_element_type=jnp.float32)

    qb = q.astype(jnp.bfloat16)
    mb = mb_ref[...]
    qm = jax.lax.dot_general(qt, m, (((0,), (0,)), ((), ())),
                             precision=hi, preferred_element_type=jnp.float32)
    msq = mstat_ref[0:1, :]
    hsq = jnp.sum(qt * qt)

    mcs = mstat_ref[1:2, :]
    _CH = 128
    _NC = _K // _CH
    kio = jax.lax.broadcasted_iota(jnp.int32, (_QT, _CH), 1)
    mv_l, ix_l, vv_l = [], [], []
    for c0 in range(0, _NC, 2):
        accs = [jnp.zeros((_QT, _CH), jnp.float32) for _ in range(2)]
        for dd in range(0, _D, 32):
            cols = [qb[:, dd + j:dd + j + 1] for j in range(32)]
            for ci in range(2):
                c = c0 + ci
                mbc = mb[:, c * _CH:(c + 1) * _CH]
                terms = [jnp.minimum(cols[j], mbc[dd + j:dd + j + 1, :])
                         for j in range(32)]
                while len(terms) > 1:
                    terms = [terms[t] + terms[t + 1]
                             for t in range(0, len(terms), 2)]
                accs[ci] = accs[ci] + terms[0].astype(jnp.float32)
        for ci in range(2):
            c = c0 + ci
            sc = mcs[:, c * _CH:(c + 1) * _CH] - 2.0 * accs[ci]
            mv = jnp.min(sc, axis=1, keepdims=True)
            ix = jnp.min(jnp.where(sc == mv, kio, _K), axis=1, keepdims=True)
            qmc = qm[:, c * _CH:(c + 1) * _CH]
            msqc = msq[:, c * _CH:(c + 1) * _CH]
            vv = jnp.sum(jnp.where(kio == ix, msqc - 2.0 * qmc, 0.0),
                         axis=1, keepdims=True)
            mv_l.append(mv)
            ix_l.append(ix + c * _CH)
            vv_l.append(vv)

    mv_all = jnp.concatenate(mv_l, axis=1)
    ix_all = jnp.concatenate(ix_l, axis=1)
    vv_all = jnp.concatenate(vv_l, axis=1)
    minv = jnp.min(mv_all, axis=1, keepdims=True)
    idx = jnp.min(jnp.where(mv_all == minv, ix_all, _K), axis=1, keepdims=True)
    val = jnp.sum(jnp.where(ix_all == idx, vv_all, 0.0), axis=1)
    msum = hsq + jnp.sum(val)

    out_ref[...] += _place(rec, 0) + _place(dsum, 1) + _place(msum, 2)

    @pl.when(i == nsteps - 1)
    def _fin():
        rg = rg_acc[...]
        g = g_acc[...] * jnp.reshape(dw_ref[...], (_D, 1))
        out_ref[...] += _place(jnp.sum(rg * rg), 3) + _place(jnp.sum(g * g), 4)


def kernel(pre_x, X, H, M, W, disc_w):
    B, T, dx = pre_x.shape
    p = pre_x.reshape(_BT, _D)
    x = X.reshape(_BT, _D)
    dw = disc_w.reshape(1, _D)
    tiles_per_b = T // _QT

    nsteps = _BT // _QT
    out = pl.pallas_call(
        _fused_step,
        grid=(nsteps,),
        in_specs=[
            pl.BlockSpec((_QT, _D), lambda i: (i, 0)),
            pl.BlockSpec((_QT, _D), lambda i: (i, 0)),
            pl.BlockSpec((1, _D, _QT),
                         lambda i: (i // tiles_per_b, 0, i % tiles_per_b)),
            pl.BlockSpec((_D, _K), lambda i: (0, 0)),
            pl.BlockSpec((_D, _D), lambda i: (0, 0)),
            pl.BlockSpec((1, _D), lambda i: (0, 0)),
        ],
        out_specs=pl.BlockSpec((8, 128), lambda i: (0, 0)),
        out_shape=jax.ShapeDtypeStruct((8, 128), jnp.float32),
        scratch_shapes=[
            pltpu.VMEM((_D, _D), jnp.float32),
            pltpu.VMEM((_D, _D), jnp.float32),
            pltpu.VMEM((_D, _K), jnp.bfloat16),
            pltpu.VMEM((8, _K), jnp.float32),
        ],
        compiler_params=pltpu.CompilerParams(
            dimension_semantics=("arbitrary",),
            vmem_limit_bytes=100 * 1024 * 1024,
        ),
    )(p, x, H, M, W, dw)

    n_rec = float(_BT * _D)
    loss_rec = out[0, 0] / n_rec
    loss_d = -out[0, 1] / float(_BT)
    loss_m = 2.0 * out[0, 2] / n_rec
    rg_norm = jnp.sqrt(out[0, 3]) * (2.0 / n_rec)
    dg_norm = jnp.sqrt(out[0, 4]) / float(_BT)
    lmbda = rg_norm / (dg_norm + _GAMMA)
    return loss_rec + _ALPHA * loss_m + lmbda * loss_d


# speedup vs baseline: 1.5677x; 1.0588x over previous
import jax
import jax.numpy as jnp
from jax.experimental import pallas as pl
from jax.experimental.pallas import tpu as pltpu

_ALPHA = 1.0
_GAMMA = 1e-06

_BT = 8192
_D = 256
_K = 512
_QT = 512


def _place(val, lane):
    r = jax.lax.broadcasted_iota(jnp.int32, (8, 128), 0)
    l = jax.lax.broadcasted_iota(jnp.int32, (8, 128), 1)
    return jnp.where((r == 0) & (l == lane), val, 0.0)


def _fused_step(p_ref, x_ref, q_ref, m_ref, w_ref, dw_ref,
                out_ref, rg_acc, g_acc, mb_ref, mstat_ref):
    i = pl.program_id(0)
    nsteps = pl.num_programs(0)

    @pl.when(i == 0)
    def _init():
        rg_acc[...] = jnp.zeros_like(rg_acc)
        g_acc[...] = jnp.zeros_like(g_acc)
        out_ref[...] = jnp.zeros_like(out_ref)
        m0 = m_ref[...]
        mb_ref[...] = m0.astype(jnp.bfloat16)
        mstat_ref[0:1, :] = jnp.sum(m0 * m0, axis=0, keepdims=True)
        mstat_ref[1:2, :] = jnp.sum(m0, axis=0, keepdims=True)

    p = p_ref[...]
    x = x_ref[...]
    qt = q_ref[0]
    q = qt.T
    m = m_ref[...]
    w = w_ref[...]
    dw = dw_ref[...]

    hi = jax.lax.Precision.HIGHEST
    y = jax.lax.dot_general(p, w, (((1,), (1,)), ((), ())),
                            precision=hi, preferred_element_type=jnp.float32)
    e = y - x
    rec = jnp.sum(e * e)
    a = jnp.tanh(y)
    dsum = jnp.sum(a * dw)
    rg_acc[...] += jax.lax.dot_general(e, p, (((0,), (0,)), ((), ())),
                                       precision=hi,
                                       preferred_element_type=jnp.float32)
    g_acc[...] += jax.lax.dot_general(1.0 - a * a, p, (((0,), (0,)), ((), ())),
                                      precision=hi,
                                      preferred_element_type=jnp.float32)

    qb = q.astype(jnp.bfloat16)
    mb = mb_ref[...]
    qm = jax.lax.dot_general(qt, m, (((0,), (0,)), ((), ())),
                             precision=hi, preferred_element_type=jnp.float32)
    msq = mstat_ref[0:1, :]
    hsq = jnp.sum(qt * qt)

    mcs = mstat_ref[1:2, :]
    _CH = 128
    _NC = _K // _CH
    kio = jax.lax.broadcasted_iota(jnp.int32, (_QT, _CH), 1)
    mv_l, ix_l, vv_l = [], [], []
    for c0 in range(0, _NC, 2):
        accs = [jnp.zeros((_QT, _CH), jnp.float32) for _ in range(2)]
        for dd in range(0, _D, 32):
            cols = [qb[:, dd + j:dd + j + 1] for j in range(32)]
            for ci in range(2):
                c = c0 + ci
                mbc = mb[:, c * _CH:(c + 1) * _CH]
                terms = [jnp.minimum(cols[j], mbc[dd + j:dd + j + 1, :])
                         for j in range(32)]
                while len(terms) > 1:
                    terms = [terms[t] + terms[t + 1]
                             for t in range(0, len(terms), 2)]
                accs[ci] = accs[ci] + terms[0].astype(jnp.float32)
        for ci in range(2):
            c = c0 + ci
            sc = mcs[:, c * _CH:(c + 1) * _CH] - 2.0 * accs[ci]
            mv = jnp.min(sc, axis=1, keepdims=True)
            ix = jnp.min(jnp.where(sc == mv, kio, _K), axis=1, keepdims=True)
            qmc = qm[:, c * _CH:(c + 1) * _CH]
            msqc = msq[:, c * _CH:(c + 1) * _CH]
            vv = jnp.sum(jnp.where(kio == ix, msqc - 2.0 * qmc, 0.0),
                         axis=1, keepdims=True)
            mv_l.append(mv)
            ix_l.append(ix + c * _CH)
            vv_l.append(vv)

    mv_all = jnp.concatenate(mv_l, axis=1)
    ix_all = jnp.concatenate(ix_l, axis=1)
    vv_all = jnp.concatenate(vv_l, axis=1)
    minv = jnp.min(mv_all, axis=1, keepdims=True)
    idx = jnp.min(jnp.where(mv_all == minv, ix_all, _K), axis=1, keepdims=True)
    val = jnp.sum(jnp.where(ix_all == idx, vv_all, 0.0), axis=1)
    msum = hsq + jnp.sum(val)

    out_ref[...] += _place(rec, 0) + _place(dsum, 1) + _place(msum, 2)

    @pl.when(i == nsteps - 1)
    def _fin():
        rg = rg_acc[...]
        g = g_acc[...] * jnp.reshape(dw_ref[...], (_D, 1))
        out_ref[...] += _place(jnp.sum(rg * rg), 3) + _place(jnp.sum(g * g), 4)


def kernel(pre_x, X, H, M, W, disc_w):
    B, T, dx = pre_x.shape
    p = pre_x.reshape(_BT, _D)
    x = X.reshape(_BT, _D)
    dw = disc_w.reshape(1, _D)
    tiles_per_b = T // _QT

    nsteps = _BT // _QT
    out = pl.pallas_call(
        _fused_step,
        grid=(nsteps,),
        in_specs=[
            pl.BlockSpec((_QT, _D), lambda i: (i, 0)),
            pl.BlockSpec((_QT, _D), lambda i: (i, 0)),
            pl.BlockSpec((1, _D, _QT),
                         lambda i: (i // tiles_per_b, 0, i % tiles_per_b)),
            pl.BlockSpec((_D, _K), lambda i: (0, 0)),
            pl.BlockSpec((_D, _D), lambda i: (0, 0)),
            pl.BlockSpec((1, _D), lambda i: (0, 0)),
        ],
        out_specs=pl.BlockSpec((8, 128), lambda i: (0, 0)),
        out_shape=jax.ShapeDtypeStruct((8, 128), jnp.float32),
        scratch_shapes=[
            pltpu.VMEM((_D, _D), jnp.float32),
            pltpu.VMEM((_D, _D), jnp.float32),
            pltpu.VMEM((_D, _K), jnp.bfloat16),
            pltpu.VMEM((8, _K), jnp.float32),
        ],
        compiler_params=pltpu.CompilerParams(
            dimension_semantics=("arbitrary",),
            vmem_limit_bytes=100 * 1024 * 1024,
        ),
    )(p, x, H, M, W, dw)

    n_rec = float(_BT * _D)
    loss_rec = out[0, 0] / n_rec
    loss_d = -out[0, 1] / float(_BT)
    loss_m = 2.0 * out[0, 2] / n_rec
    rg_norm = jnp.sqrt(out[0, 3]) * (2.0 / n_rec)
    dg_norm = jnp.sqrt(out[0, 4]) / float(_BT)
    lmbda = rg_norm / (dg_norm + _GAMMA)
    return loss_rec + _ALPHA * loss_m + lmbda * loss_d


# speedup vs baseline: 1.6105x; 1.0274x over previous
import jax
import jax.numpy as jnp
from jax.experimental import pallas as pl
from jax.experimental.pallas import tpu as pltpu

_ALPHA = 1.0
_GAMMA = 1e-06

_BT = 8192
_D = 256
_K = 512
_QT = 1024


def _place(val, lane):
    r = jax.lax.broadcasted_iota(jnp.int32, (8, 128), 0)
    l = jax.lax.broadcasted_iota(jnp.int32, (8, 128), 1)
    return jnp.where((r == 0) & (l == lane), val, 0.0)


def _fused_step(p_ref, x_ref, q_ref, m_ref, w_ref, dw_ref,
                out_ref, rg_acc, g_acc, mb_ref, mstat_ref):
    i = pl.program_id(0)
    nsteps = pl.num_programs(0)

    @pl.when(i == 0)
    def _init():
        rg_acc[...] = jnp.zeros_like(rg_acc)
        g_acc[...] = jnp.zeros_like(g_acc)
        out_ref[...] = jnp.zeros_like(out_ref)
        m0 = m_ref[...]
        mb_ref[...] = m0.astype(jnp.bfloat16)
        mstat_ref[0:1, :] = jnp.sum(m0 * m0, axis=0, keepdims=True)
        mstat_ref[1:2, :] = jnp.sum(m0, axis=0, keepdims=True)

    p = p_ref[...]
    x = x_ref[...]
    qt = jnp.concatenate([q_ref[0], q_ref[1]], axis=1)
    q = qt.T
    m = m_ref[...]
    w = w_ref[...]
    dw = dw_ref[...]

    hi = jax.lax.Precision.HIGHEST
    y = jax.lax.dot_general(p, w, (((1,), (1,)), ((), ())),
                            precision=hi, preferred_element_type=jnp.float32)
    e = y - x
    rec = jnp.sum(e * e)
    a = jnp.tanh(y)
    dsum = jnp.sum(a * dw)
    rg_acc[...] += jax.lax.dot_general(e, p, (((0,), (0,)), ((), ())),
                                       precision=hi,
                                       preferred_element_type=jnp.float32)
    g_acc[...] += jax.lax.dot_general(1.0 - a * a, p, (((0,), (0,)), ((), ())),
                                      precision=hi,
                                      preferred_element_type=jnp.float32)

    qb = q.astype(jnp.bfloat16)
    mb = mb_ref[...]
    qm = jax.lax.dot_general(qt, m, (((0,), (0,)), ((), ())),
                             precision=hi, preferred_element_type=jnp.float32)
    msq = mstat_ref[0:1, :]
    hsq = jnp.sum(qt * qt)

    mcs = mstat_ref[1:2, :]
    _CH = 128
    _NC = _K // _CH
    kio = jax.lax.broadcasted_iota(jnp.int32, (_QT, _CH), 1)
    mv_l, ix_l, vv_l = [], [], []
    for c0 in range(0, _NC, 2):
        accs = [jnp.zeros((_QT, _CH), jnp.float32) for _ in range(2)]
        for dd in range(0, _D, 32):
            cols = [qb[:, dd + j:dd + j + 1] for j in range(32)]
            for ci in range(2):
                c = c0 + ci
                mbc = mb[:, c * _CH:(c + 1) * _CH]
                terms = [jnp.minimum(cols[j], mbc[dd + j:dd + j + 1, :])
                         for j in range(32)]
                while len(terms) > 1:
                    terms = [terms[t] + terms[t + 1]
                             for t in range(0, len(terms), 2)]
                accs[ci] = accs[ci] + terms[0].astype(jnp.float32)
        for ci in range(2):
            c = c0 + ci
            sc = mcs[:, c * _CH:(c + 1) * _CH] - 2.0 * accs[ci]
            mv = jnp.min(sc, axis=1, keepdims=True)
            ix = jnp.min(jnp.where(sc == mv, kio, _K), axis=1, keepdims=True)
            qmc = qm[:, c * _CH:(c + 1) * _CH]
            msqc = msq[:, c * _CH:(c + 1) * _CH]
            vv = jnp.sum(jnp.where(kio == ix, msqc - 2.0 * qmc, 0.0),
                         axis=1, keepdims=True)
            mv_l.append(mv)
            ix_l.append(ix + c * _CH)
            vv_l.append(vv)

    mv_all = jnp.concatenate(mv_l, axis=1)
    ix_all = jnp.concatenate(ix_l, axis=1)
    vv_all = jnp.concatenate(vv_l, axis=1)
    minv = jnp.min(mv_all, axis=1, keepdims=True)
    idx = jnp.min(jnp.where(mv_all == minv, ix_all, _K), axis=1, keepdims=True)
    val = jnp.sum(jnp.where(ix_all == idx, vv_all, 0.0), axis=1)
    msum = hsq + jnp.sum(val)

    out_ref[...] += _place(rec, 0) + _place(dsum, 1) + _place(msum, 2)

    @pl.when(i == nsteps - 1)
    def _fin():
        rg = rg_acc[...]
        g = g_acc[...] * jnp.reshape(dw_ref[...], (_D, 1))
        out_ref[...] += _place(jnp.sum(rg * rg), 3) + _place(jnp.sum(g * g), 4)


def kernel(pre_x, X, H, M, W, disc_w):
    B, T, dx = pre_x.shape
    p = pre_x.reshape(_BT, _D)
    x = X.reshape(_BT, _D)
    dw = disc_w.reshape(1, _D)
    tiles_per_b = T // _QT

    nsteps = _BT // _QT
    out = pl.pallas_call(
        _fused_step,
        grid=(nsteps,),
        in_specs=[
            pl.BlockSpec((_QT, _D), lambda i: (i, 0)),
            pl.BlockSpec((_QT, _D), lambda i: (i, 0)),
            pl.BlockSpec((2, _D, 512), lambda i: (i, 0, 0)),
            pl.BlockSpec((_D, _K), lambda i: (0, 0)),
            pl.BlockSpec((_D, _D), lambda i: (0, 0)),
            pl.BlockSpec((1, _D), lambda i: (0, 0)),
        ],
        out_specs=pl.BlockSpec((8, 128), lambda i: (0, 0)),
        out_shape=jax.ShapeDtypeStruct((8, 128), jnp.float32),
        scratch_shapes=[
            pltpu.VMEM((_D, _D), jnp.float32),
            pltpu.VMEM((_D, _D), jnp.float32),
            pltpu.VMEM((_D, _K), jnp.bfloat16),
            pltpu.VMEM((8, _K), jnp.float32),
        ],
        compiler_params=pltpu.CompilerParams(
            dimension_semantics=("arbitrary",),
            vmem_limit_bytes=100 * 1024 * 1024,
        ),
    )(p, x, H, M, W, dw)

    n_rec = float(_BT * _D)
    loss_rec = out[0, 0] / n_rec
    loss_d = -out[0, 1] / float(_BT)
    loss_m = 2.0 * out[0, 2] / n_rec
    rg_norm = jnp.sqrt(out[0, 3]) * (2.0 / n_rec)
    dg_norm = jnp.sqrt(out[0, 4]) / float(_BT)
    lmbda = rg_norm / (dg_norm + _GAMMA)
    return loss_rec + _ALPHA * loss_m + lmbda * loss_d
